# baseline (speedup 1.0000x reference)
import jax
import jax.numpy as jnp
from jax import lax
from jax.experimental import pallas as pl
from jax.experimental.pallas import tpu as pltpu
from jax.experimental.pallas import tpu_sc as plsc

B, D = 16384, 64
L = 16

NC, NS = 2, 16
NW = NC * NS
SC_N = 4096
CPW = SC_N // NW
CH = 128
NCH = CPW // CH

BC = 2048
TC_STEPS = (B - SC_N) // BC


def _sc_body(u_hbm, ub_hbm, i_hbm, ib_hbm, out_hbm,
             u0_v, u1_v, i0_v, i1_v, ub_v, ib_v, out_v,
             sem_u0, sem_u1, sem_i0, sem_i1):
    wid = lax.axis_index("s") * NC + lax.axis_index("c")
    base = wid * CPW
    u_bufs, i_bufs = (u0_v, u1_v), (i0_v, i1_v)
    sem_us, sem_is = (sem_u0, sem_u1), (sem_i0, sem_i1)

    def start(c):
        b = c % 2
        cu = pltpu.async_copy(
            u_hbm.at[:, pl.ds(base + c * CH, CH)], u_bufs[b], sem_us[b])
        ci = pltpu.async_copy(
            i_hbm.at[:, pl.ds(base + c * CH, CH)], i_bufs[b], sem_is[b])
        return cu, ci

    inflight = start(0)
    pltpu.sync_copy(ub_hbm.at[pl.ds(base, CPW)], ub_v)
    pltpu.sync_copy(ib_hbm.at[pl.ds(base, CPW)], ib_v)

    for c in range(NCH):
        cu, ci = inflight
        if c + 1 < NCH:
            inflight = start(c + 1)
        cu.wait()
        ci.wait()
        u_v, i_v = u_bufs[c % 2], i_bufs[c % 2]

        def group(g, _, u_v=u_v, i_v=i_v, c=c):
            col = g * L
            acc = [u_v[k, pl.ds(col, L)] * i_v[k, pl.ds(col, L)]
                   for k in range(4)]
            for d in range(4, D):
                acc[d % 4] = acc[d % 4] + (
                    u_v[d, pl.ds(col, L)] * i_v[d, pl.ds(col, L)])
            a0 = c * CH + col
            out_v[pl.ds(a0, L)] = (
                ((acc[0] + acc[1]) + (acc[2] + acc[3]))
                + (ub_v[pl.ds(a0, L)] + ib_v[pl.ds(a0, L)]))
            return 0

        lax.fori_loop(0, CH // L, group, 0)

    pltpu.sync_copy(out_v, out_hbm.at[pl.ds(base, CPW)])


def _tc_body(u_ref, i_ref, ub_ref, ib_ref, o_ref):
    dot = jnp.sum(u_ref[...] * i_ref[...], axis=0)
    o_ref[...] = dot + ub_ref[...] + ib_ref[...]


def kernel(user_representation, user_bias, item_representation, item_bias):
    ut = user_representation.T
    it = item_representation.T

    mesh = plsc.VectorSubcoreMesh(
        core_axis_name="c", subcore_axis_name="s", num_cores=NC)
    sc_fn = pl.kernel(
        _sc_body,
        mesh=mesh,
        out_type=jax.ShapeDtypeStruct((SC_N,), jnp.float32),
        compiler_params=pltpu.CompilerParams(needs_layout_passes=False),
        scratch_types=[
            pltpu.VMEM((D, CH), jnp.float32),
            pltpu.VMEM((D, CH), jnp.float32),
            pltpu.VMEM((D, CH), jnp.float32),
            pltpu.VMEM((D, CH), jnp.float32),
            pltpu.VMEM((CPW,), jnp.float32),
            pltpu.VMEM((CPW,), jnp.float32),
            pltpu.VMEM((CPW,), jnp.float32),
            pltpu.SemaphoreType.DMA,
            pltpu.SemaphoreType.DMA,
            pltpu.SemaphoreType.DMA,
            pltpu.SemaphoreType.DMA,
        ],
    )
    sc_out = sc_fn(ut, user_bias, it, item_bias)

    off = SC_N // BC
    tc_out = pl.pallas_call(
        _tc_body,
        grid=(TC_STEPS,),
        in_specs=[
            pl.BlockSpec((D, BC), lambda j: (0, off + j)),
            pl.BlockSpec((D, BC), lambda j: (0, off + j)),
            pl.BlockSpec((BC,), lambda j: (off + j,)),
            pl.BlockSpec((BC,), lambda j: (off + j,)),
        ],
        out_specs=pl.BlockSpec((BC,), lambda j: (j,)),
        out_shape=jax.ShapeDtypeStruct((B - SC_N,), jnp.float32),
    )(ut, it, user_bias, item_bias)

    out = jnp.zeros((B,), jnp.float32)
    out = lax.dynamic_update_slice(out, sc_out, (0,))
    out = lax.dynamic_update_slice(out, tc_out, (SC_N,))
    return out


# speedup vs baseline: 1.0612x; 1.0612x over previous
import jax
import jax.numpy as jnp
from jax import lax
from jax.experimental import pallas as pl
from jax.experimental.pallas import tpu as pltpu
from jax.experimental.pallas import tpu_sc as plsc

B, D = 16384, 64
L = 16

NC, NS = 2, 16
NW = NC * NS
SC_N = 4096
CPW = SC_N // NW
CH = 128
NCH = CPW // CH

BC = 2048
TC_STEPS = (B - SC_N) // BC


DH = D // 2


def _sc_body(u_hbm, ub_hbm, i_hbm, ib_hbm, out_hbm,
             u0_v, u1_v, i0_v, i1_v, ub_v, ib_v, out_v,
             sem_u0, sem_u1, sem_i0, sem_i1):
    wid = lax.axis_index("s") * NC + lax.axis_index("c")
    base = wid * CPW

    cu0 = pltpu.async_copy(
        u_hbm.at[pl.ds(0, DH), pl.ds(base, CPW)], u0_v, sem_u0)
    ci0 = pltpu.async_copy(
        i_hbm.at[pl.ds(0, DH), pl.ds(base, CPW)], i0_v, sem_i0)
    cu1 = pltpu.async_copy(
        u_hbm.at[pl.ds(DH, DH), pl.ds(base, CPW)], u1_v, sem_u1)
    ci1 = pltpu.async_copy(
        i_hbm.at[pl.ds(DH, DH), pl.ds(base, CPW)], i1_v, sem_i1)
    pltpu.sync_copy(ub_hbm.at[pl.ds(base, CPW)], ub_v)
    pltpu.sync_copy(ib_hbm.at[pl.ds(base, CPW)], ib_v)

    def phase(u_v, i_v, first):
        def group(g, _):
            col = g * L
            acc = [u_v[k, pl.ds(col, L)] * i_v[k, pl.ds(col, L)]
                   for k in range(4)]
            for d in range(4, DH):
                acc[d % 4] = acc[d % 4] + (
                    u_v[d, pl.ds(col, L)] * i_v[d, pl.ds(col, L)])
            s = (acc[0] + acc[1]) + (acc[2] + acc[3])
            if first:
                out_v[pl.ds(col, L)] = (
                    s + (ub_v[pl.ds(col, L)] + ib_v[pl.ds(col, L)]))
            else:
                out_v[pl.ds(col, L)] = out_v[pl.ds(col, L)] + s
            return 0
        lax.fori_loop(0, CPW // L, group, 0)

    cu0.wait()
    ci0.wait()
    phase(u0_v, i0_v, True)
    cu1.wait()
    ci1.wait()
    phase(u1_v, i1_v, False)

    pltpu.sync_copy(out_v, out_hbm.at[pl.ds(base, CPW)])


def _tc_body(u_ref, i_ref, ub_ref, ib_ref, o_ref):
    dot = jnp.sum(u_ref[...] * i_ref[...], axis=0)
    o_ref[...] = dot + ub_ref[...] + ib_ref[...]


def kernel(user_representation, user_bias, item_representation, item_bias):
    ut = user_representation.T
    it = item_representation.T

    mesh = plsc.VectorSubcoreMesh(
        core_axis_name="c", subcore_axis_name="s", num_cores=NC)
    sc_fn = pl.kernel(
        _sc_body,
        mesh=mesh,
        out_type=jax.ShapeDtypeStruct((SC_N,), jnp.float32),
        compiler_params=pltpu.CompilerParams(needs_layout_passes=False),
        scratch_types=[
            pltpu.VMEM((DH, CPW), jnp.float32),
            pltpu.VMEM((DH, CPW), jnp.float32),
            pltpu.VMEM((DH, CPW), jnp.float32),
            pltpu.VMEM((DH, CPW), jnp.float32),
            pltpu.VMEM((CPW,), jnp.float32),
            pltpu.VMEM((CPW,), jnp.float32),
            pltpu.VMEM((CPW,), jnp.float32),
            pltpu.SemaphoreType.DMA,
            pltpu.SemaphoreType.DMA,
            pltpu.SemaphoreType.DMA,
            pltpu.SemaphoreType.DMA,
        ],
    )
    sc_out = sc_fn(ut, user_bias, it, item_bias)

    off = SC_N // BC
    tc_out = pl.pallas_call(
        _tc_body,
        grid=(TC_STEPS,),
        in_specs=[
            pl.BlockSpec((D, BC), lambda j: (0, off + j)),
            pl.BlockSpec((D, BC), lambda j: (0, off + j)),
            pl.BlockSpec((BC,), lambda j: (off + j,)),
            pl.BlockSpec((BC,), lambda j: (off + j,)),
        ],
        out_specs=pl.BlockSpec((BC,), lambda j: (j,)),
        out_shape=jax.ShapeDtypeStruct((B - SC_N,), jnp.float32),
    )(ut, it, user_bias, item_bias)

    return jnp.concatenate([sc_out, tc_out])


# speedup vs baseline: 1.0633x; 1.0020x over previous
import jax
import jax.numpy as jnp
from jax import lax
from jax.experimental import pallas as pl
from jax.experimental.pallas import tpu as pltpu
from jax.experimental.pallas import tpu_sc as plsc

B, D = 16384, 64
L = 16

NC, NS = 2, 16
NW = NC * NS
SC_N = 4096
CPW = SC_N // NW
CH = 128
NCH = CPW // CH

BC = 2048
TC_STEPS = (B - SC_N) // BC


DH = D // 2


def _sc_body(u_hbm, ub_hbm, i_hbm, ib_hbm, out_hbm,
             u0_v, u1_v, i0_v, i1_v, ub_v, ib_v, out_v,
             sem_u0, sem_u1, sem_i0, sem_i1):
    wid = lax.axis_index("s") * NC + lax.axis_index("c")
    base = wid * CPW

    cu0 = pltpu.async_copy(
        u_hbm.at[pl.ds(0, DH), pl.ds(base, CPW)], u0_v, sem_u0)
    ci0 = pltpu.async_copy(
        i_hbm.at[pl.ds(0, DH), pl.ds(base, CPW)], i0_v, sem_i0)
    cu1 = pltpu.async_copy(
        u_hbm.at[pl.ds(DH, DH), pl.ds(base, CPW)], u1_v, sem_u1)
    ci1 = pltpu.async_copy(
        i_hbm.at[pl.ds(DH, DH), pl.ds(base, CPW)], i1_v, sem_i1)
    pltpu.sync_copy(ub_hbm.at[pl.ds(base, CPW)], ub_v)
    pltpu.sync_copy(ib_hbm.at[pl.ds(base, CPW)], ib_v)

    def phase(u_v, i_v, first):
        def group(g, _):
            col = g * L
            acc = [u_v[k, pl.ds(col, L)] * i_v[k, pl.ds(col, L)]
                   for k in range(4)]
            for d in range(4, DH):
                acc[d % 4] = acc[d % 4] + (
                    u_v[d, pl.ds(col, L)] * i_v[d, pl.ds(col, L)])
            s = (acc[0] + acc[1]) + (acc[2] + acc[3])
            if first:
                out_v[pl.ds(col, L)] = (
                    s + (ub_v[pl.ds(col, L)] + ib_v[pl.ds(col, L)]))
            else:
                out_v[pl.ds(col, L)] = out_v[pl.ds(col, L)] + s
            return 0
        lax.fori_loop(0, CPW // L, group, 0)

    cu0.wait()
    ci0.wait()
    phase(u0_v, i0_v, True)
    cu1.wait()
    ci1.wait()
    phase(u1_v, i1_v, False)

    pltpu.sync_copy(out_v, out_hbm.at[pl.ds(base, CPW)])


def _tc_body(u_ref, i_ref, ub_ref, ib_ref, o_ref):
    dot = jnp.sum(u_ref[...] * i_ref[...], axis=0)
    o_ref[...] = dot + ub_ref[...] + ib_ref[...]


def kernel(user_representation, user_bias, item_representation, item_bias):
    ut = user_representation.T
    it = item_representation.T

    mesh = plsc.VectorSubcoreMesh(
        core_axis_name="c", subcore_axis_name="s", num_cores=NC)
    sc_fn = pl.kernel(
        _sc_body,
        mesh=mesh,
        out_type=jax.ShapeDtypeStruct((SC_N,), jnp.float32),
        compiler_params=pltpu.CompilerParams(needs_layout_passes=False),
        scratch_types=[
            pltpu.VMEM((DH, CPW), jnp.float32),
            pltpu.VMEM((DH, CPW), jnp.float32),
            pltpu.VMEM((DH, CPW), jnp.float32),
            pltpu.VMEM((DH, CPW), jnp.float32),
            pltpu.VMEM((CPW,), jnp.float32),
            pltpu.VMEM((CPW,), jnp.float32),
            pltpu.VMEM((CPW,), jnp.float32),
            pltpu.SemaphoreType.DMA,
            pltpu.SemaphoreType.DMA,
            pltpu.SemaphoreType.DMA,
            pltpu.SemaphoreType.DMA,
        ],
    )
    off = SC_N // BC
    tc_out = pl.pallas_call(
        _tc_body,
        grid=(TC_STEPS,),
        in_specs=[
            pl.BlockSpec((D, BC), lambda j: (0, off + j)),
            pl.BlockSpec((D, BC), lambda j: (0, off + j)),
            pl.BlockSpec((BC,), lambda j: (off + j,)),
            pl.BlockSpec((BC,), lambda j: (off + j,)),
        ],
        out_specs=pl.BlockSpec((BC,), lambda j: (j,)),
        out_shape=jax.ShapeDtypeStruct((B - SC_N,), jnp.float32),
    )(ut, it, user_bias, item_bias)

    sc_out = sc_fn(ut, user_bias, it, item_bias)
    return jnp.concatenate([sc_out, tc_out])
